# baseline (speedup 1.0000x reference)
import jax
import jax.numpy as jnp
from jax.experimental import pallas as pl

_N = 1000
_HW = 104 * 104
_S = _HW // 8

def _tiny(b_ref, s_ref, out_ref):
    out_ref[...] = s_ref[...] * 2.0 + jnp.sum(b_ref[...])

def kernel(seg_masks_soft, cate_labels, cate_scores):
    w = (2.0 ** jnp.arange(8, dtype=jnp.float32)).reshape(1, 1, 8)
    p = ((seg_masks_soft.reshape(_N, _S, 8) > 0.005).astype(jnp.float32) * w).sum(axis=2)
    scores = cate_scores.reshape(1, _N)
    out = pl.pallas_call(
        _tiny,
        in_specs=[
            pl.BlockSpec((32, 128), lambda i: (0, 0)),
            pl.BlockSpec((1, _N), lambda i: (0, 0)),
        ],
        out_specs=pl.BlockSpec((1, _N), lambda i: (0, 0)),
        out_shape=jax.ShapeDtypeStruct((1, _N), jnp.float32),
        grid=(1,),
    )(p, scores)
    return out[0]


# speedup vs baseline: 1.2051x; 1.2051x over previous
import jax
import jax.numpy as jnp
from jax.experimental import pallas as pl
from jax.experimental.pallas import tpu as pltpu

_N = 1000
_HW = 104 * 104
_S = _HW // 8
_MASK_THR = 0.005
_SIGMA = 2.0


def _nms_kernel(p_ref, labels_ref, scores_ref, out_ref):
    p = p_ref[...].astype(jnp.int32)
    inter = None
    for q in range(8):
        bq = ((p >> q) & 1).astype(jnp.bfloat16)
        pq = jax.lax.dot_general(
            bq, bq, (((1,), (1,)), ((), ())), preferred_element_type=jnp.float32
        )
        inter = pq if inter is None else inter + pq

    i_idx = jax.lax.broadcasted_iota(jnp.int32, (_N, _N), 0)
    j_idx = jax.lax.broadcasted_iota(jnp.int32, (_N, _N), 1)
    s_row = jnp.sum(jnp.where(i_idx == j_idx, inter, 0.0), axis=0, keepdims=True)
    s_col = s_row.reshape(_N, 1)
    lab_row = labels_ref[...]
    lab_col = lab_row.reshape(_N, 1)
    mask = (i_idx < j_idx) & (lab_col == lab_row)
    d = jnp.where(mask, inter / (s_col + s_row - inter), 0.0)
    comp_row = jnp.max(d, axis=0, keepdims=True)
    comp_col = comp_row.reshape(_N, 1)
    m = jnp.max(d * d - comp_col * comp_col, axis=0, keepdims=True)
    out_ref[...] = scores_ref[...] * jnp.exp(-_SIGMA * m)


def kernel(seg_masks_soft, cate_labels, cate_scores):
    w = (2.0 ** jnp.arange(8, dtype=jnp.float32)).reshape(1, 8, 1)
    p = ((seg_masks_soft.reshape(_N, 8, _S) > _MASK_THR).astype(jnp.float32) * w).sum(axis=1)
    labels = cate_labels.reshape(1, _N)
    scores = cate_scores.reshape(1, _N)
    out = pl.pallas_call(
        _nms_kernel,
        grid=(1,),
        in_specs=[
            pl.BlockSpec((_N, _S), lambda i: (0, 0)),
            pl.BlockSpec((1, _N), lambda i: (0, 0)),
            pl.BlockSpec((1, _N), lambda i: (0, 0)),
        ],
        out_specs=pl.BlockSpec((1, _N), lambda i: (0, 0)),
        out_shape=jax.ShapeDtypeStruct((1, _N), jnp.float32),
        compiler_params=pltpu.CompilerParams(vmem_limit_bytes=128 * 1024 * 1024),
    )(p, labels, scores)
    return out[0]


# speedup vs baseline: 1.3165x; 1.0925x over previous
import jax
import jax.numpy as jnp
from jax.experimental import pallas as pl
from jax.experimental.pallas import tpu as pltpu

_N = 1000
_HW = 104 * 104
_S = _HW // 8
_MASK_THR = 0.005
_SIGMA = 2.0


def _nms_kernel(p_ref, labels_ref, scores_ref, out_ref):
    p = p_ref[...].astype(jnp.int32)
    inter = None
    for q in range(8):
        bq = ((p >> q) & 1).astype(jnp.bfloat16)
        pq = jax.lax.dot_general(
            bq, bq, (((1,), (1,)), ((), ())), preferred_element_type=jnp.float32
        )
        inter = pq if inter is None else inter + pq

    i_idx = jax.lax.broadcasted_iota(jnp.int32, (_N, _N), 0)
    j_idx = jax.lax.broadcasted_iota(jnp.int32, (_N, _N), 1)
    s_row = jnp.sum(jnp.where(i_idx == j_idx, inter, 0.0), axis=0, keepdims=True)
    s_col = s_row.reshape(_N, 1)
    lab_row = labels_ref[...]
    lab_col = lab_row.reshape(_N, 1)
    mask = (i_idx < j_idx) & (lab_col == lab_row)
    d = jnp.where(mask, inter / (s_col + s_row - inter), 0.0)
    comp_row = jnp.max(d, axis=0, keepdims=True)
    comp_col = comp_row.reshape(_N, 1)
    m = jnp.max(d * d - comp_col * comp_col, axis=0, keepdims=True)
    out_ref[...] = scores_ref[...] * jnp.exp(-_SIGMA * m)


def kernel(seg_masks_soft, cate_labels, cate_scores):
    w = (2.0 ** jnp.arange(8, dtype=jnp.float32)).reshape(1, 8, 1)
    p = ((seg_masks_soft.reshape(_N, 8, _S) > _MASK_THR).astype(jnp.float32) * w).sum(axis=1).astype(jnp.bfloat16)
    labels = cate_labels.reshape(1, _N)
    scores = cate_scores.reshape(1, _N)
    out = pl.pallas_call(
        _nms_kernel,
        grid=(1,),
        in_specs=[
            pl.BlockSpec((_N, _S), lambda i: (0, 0)),
            pl.BlockSpec((1, _N), lambda i: (0, 0)),
            pl.BlockSpec((1, _N), lambda i: (0, 0)),
        ],
        out_specs=pl.BlockSpec((1, _N), lambda i: (0, 0)),
        out_shape=jax.ShapeDtypeStruct((1, _N), jnp.float32),
        compiler_params=pltpu.CompilerParams(vmem_limit_bytes=128 * 1024 * 1024),
    )(p, labels, scores)
    return out[0]
